# baseline (speedup 1.0000x reference)
import functools

import jax
import jax.numpy as jnp
from jax import lax
from jax.experimental import pallas as pl
from jax.experimental.pallas import tpu as pltpu
from jax.experimental.pallas import tpu_sc as plsc

N = 10000
E = 320000
NC, NS = 2, 16
NW = NC * NS
EPW = E // NW
CP = 128
EPWP = 10240
NCHP = EPWP // CP
NP = 10240
RPT = NP // NS
NBUF = 4
NOUT = 2


def _sc_layer(D):
    D2 = D // 2
    mesh = plsc.VectorSubcoreMesh(core_axis_name="c", subcore_axis_name="s")

    @functools.partial(
        pl.kernel,
        out_type=[
            jax.ShapeDtypeStruct((NC, NP), jnp.float32),
            jax.ShapeDtypeStruct((NC, NP, D2), jnp.float32),
        ],
        mesh=mesh,
        compiler_params=pltpu.CompilerParams(
            needs_layout_passes=False, use_tc_tiling_on_sc=False),
        scratch_types=[
            pltpu.VMEM((NCHP, CP), jnp.int32),
            pltpu.VMEM((NCHP, CP), jnp.int32),
            pltpu.VMEM((NP,), jnp.float32),
            pltpu.VMEM((NP,), jnp.float32),
            pltpu.VMEM((NOUT, CP), jnp.float32),
            pltpu.VMEM((NBUF, CP, D2), jnp.float32),
            pltpu.VMEM((NOUT, CP, D2), jnp.float32),
            pltpu.VMEM((RPT,), jnp.float32),
            pltpu.VMEM_SHARED((NP,), jnp.float32),
            pltpu.VMEM_SHARED((NP, D2), jnp.float32),
            pltpu.VMEM_SHARED((NP, D2), jnp.float32),
            pltpu.SemaphoreType.DMA((NBUF,)),
            pltpu.SemaphoreType.DMA((NOUT,)),
            pltpu.SemaphoreType.DMA((NOUT,)),
            pltpu.SemaphoreType.DMA,
        ],
    )
    def k(src_hbm, dst_hbm, s_hbm, d_hbm, h_hbm, den_out, out_out,
          srcv, dstv, sv, dv, exb, rows, rows_out, zrow, dacc, oacc, hbuf,
          gsem, ssem, dsem, hsem):
        cid = lax.axis_index("c")
        sid = lax.axis_index("s")

        zero = jnp.zeros((16,), jnp.float32)
        zidx = jnp.zeros((16,), jnp.int32)
        oidx = jnp.ones((16,), jnp.int32)

        pltpu.async_copy(s_hbm, sv, gsem.at[2])
        pltpu.async_copy(d_hbm, dv, gsem.at[3])
        pltpu.async_copy(h_hbm.at[cid, pl.ds(sid * RPT, RPT)],
                         hbuf.at[pl.ds(sid * RPT, RPT)], hsem)

        @pl.loop(0, RPT // 16, unroll=8)
        def _(i):
            zrow[pl.ds(i * 16, 16)] = zero

        @pl.loop(0, CP, unroll=4)
        def _(r):
            for t in range(D2 // 16):
                rows_out[0, r, pl.ds(t * 16, 16)] = zero

        pltpu.sync_copy(zrow, dacc.at[pl.ds(sid * RPT, RPT)])
        for t in range(RPT // CP):
            pltpu.sync_copy(rows_out.at[0],
                            oacc.at[pl.ds(sid * RPT + t * CP, CP)])

        pltpu.make_async_copy(s_hbm, sv, gsem.at[2]).wait()
        pltpu.make_async_copy(d_hbm, dv, gsem.at[3]).wait()
        pltpu.make_async_copy(h_hbm.at[cid, pl.ds(sid * RPT, RPT)],
                              hbuf.at[pl.ds(sid * RPT, RPT)], hsem).wait()
        plsc.subcore_barrier()

        for p in range(2):
            wp = sid * 2 + p
            do_den = cid == p
            pltpu.async_copy(src_hbm.at[wp], srcv, gsem.at[0])
            pltpu.async_copy(dst_hbm.at[wp], dstv, gsem.at[1])
            pltpu.make_async_copy(src_hbm.at[wp], srcv, gsem.at[0]).wait()
            pltpu.make_async_copy(dst_hbm.at[wp], dstv, gsem.at[1]).wait()

            for c in range(2):
                pltpu.async_copy(hbuf.at[srcv.at[c]], rows.at[c], gsem.at[c])

            @pl.loop(0, NCHP, step=NBUF)
            def _(cbase):
                for b in range(NBUF):
                    c = cbase + b
                    b2 = b % NOUT
                    bn = (b + 2) % NBUF

                    @pl.when(c < NCHP - 2)
                    def _():
                        pltpu.async_copy(
                            hbuf.at[srcv.at[c + 2]], rows.at[bn],
                            gsem.at[bn])

                    pltpu.make_async_copy(
                        hbuf.at[srcv.at[c]], rows.at[b], gsem.at[b]).wait()

                    @pl.when(c >= NOUT)
                    def _():
                        pltpu.make_async_copy(
                            rows_out.at[b2], oacc.at[dstv.at[c - NOUT]],
                            ssem.at[b2]).wait()

                        @pl.when(do_den)
                        def _():
                            pltpu.make_async_copy(
                                exb.at[b2],
                                dacc.at[dstv.at[c - NOUT]],
                                dsem.at[b2]).wait()

                    @plsc.parallel_loop(0, CP // 16, unroll=1)
                    def _(q):
                        si = srcv[c, pl.ds(q * 16, 16)]
                        di = dstv[c, pl.ds(q * 16, 16)]
                        e = (plsc.load_gather(sv, [si])
                             + plsc.load_gather(dv, [di]))
                        e = jnp.where(e >= 0.0, e, 0.2 * e)
                        avec = jnp.exp(e)
                        exb[b2, pl.ds(q * 16, 16)] = avec
                        for rl in range(16):
                            r = q * 16 + rl
                            a = avec.at[
                                jnp.full((16,), rl, jnp.int32)].get(
                                    mode="promise_in_bounds")
                            for t in range(D2 // 16):
                                rows_out[b2, r, pl.ds(t * 16, 16)] = (
                                    rows[b, r, pl.ds(t * 16, 16)] * a)

                    pltpu.async_copy(
                        rows_out.at[b2], oacc.at[dstv.at[c]], ssem.at[b2],
                        add=True)

                    @pl.when(do_den)
                    def _():
                        pltpu.async_copy(
                            exb.at[b2], dacc.at[dstv.at[c]],
                            dsem.at[b2], add=True)

            for c in range(NCHP - NOUT, NCHP):
                b2 = c % NOUT
                pltpu.make_async_copy(
                    rows_out.at[b2], oacc.at[dstv.at[c]], ssem.at[b2]).wait()

                @pl.when(do_den)
                def _():
                    pltpu.make_async_copy(
                        exb.at[b2], dacc.at[dstv.at[c]], dsem.at[b2]).wait()

        plsc.subcore_barrier()

        pltpu.sync_copy(dacc.at[pl.ds(sid * RPT, RPT)],
                        den_out.at[cid, pl.ds(sid * RPT, RPT)])
        pltpu.sync_copy(oacc.at[pl.ds(sid * RPT, RPT)],
                        out_out.at[cid, pl.ds(sid * RPT, RPT)])

    return k


_sc_layer1 = _sc_layer(64)
_sc_layer2 = _sc_layer(32)


def _tc_front(x, W1, a1s, a1d):
    def body(x_ref, w_ref, as_ref, ad_ref, h_ref, sd_ref):
        h = jnp.dot(x_ref[...], w_ref[...], preferred_element_type=jnp.float32)
        hp = jnp.concatenate(
            [h, jnp.zeros((NP - N, 64), jnp.float32)], axis=0)
        h_ref[0] = hp[:, :32]
        h_ref[1] = hp[:, 32:]
        a = jnp.concatenate(
            [as_ref[...][:, None], ad_ref[...][:, None]], axis=1)
        sd_ref[...] = jnp.dot(hp, a, preferred_element_type=jnp.float32)

    return pl.pallas_call(
        body,
        out_shape=[
            jax.ShapeDtypeStruct((NC, NP, 32), jnp.float32),
            jax.ShapeDtypeStruct((NP, 2), jnp.float32),
        ],
    )(x, W1, a1s, a1d)


def _tc_mid(op1, dp1, b1, W2, a2s, a2d):
    def body(op_ref, dp_ref, b_ref, w_ref, as_ref, ad_ref, h_ref, sd_ref):
        acc = jnp.concatenate([op_ref[0], op_ref[1]], axis=1)
        den = dp_ref[0] + dp_ref[1]
        rden = 1.0 / (den + 1e-16)
        xx = acc * rden[:, None] + b_ref[...][None, :]
        xx = jnp.where(xx > 0.0, xx, jnp.exp(xx) - 1.0)
        h2 = jnp.dot(xx, w_ref[...], preferred_element_type=jnp.float32)
        h_ref[0] = h2[:, :16]
        h_ref[1] = h2[:, 16:]
        a = jnp.concatenate(
            [as_ref[...][:, None], ad_ref[...][:, None]], axis=1)
        sd_ref[...] = jnp.dot(h2, a, preferred_element_type=jnp.float32)

    return pl.pallas_call(
        body,
        out_shape=[
            jax.ShapeDtypeStruct((NC, NP, 16), jnp.float32),
            jax.ShapeDtypeStruct((NP, 2), jnp.float32),
        ],
    )(op1, dp1, b1, W2, a2s, a2d)


def _tc_back(op2, dp2, b2):
    def body(op_ref, dp_ref, b_ref, o_ref):
        acc = jnp.concatenate([op_ref[0], op_ref[1]], axis=1)[:N]
        den = (dp_ref[0] + dp_ref[1])[:N]
        rden = 1.0 / (den + 1e-16)
        o_ref[...] = acc * rden[:, None] + b_ref[...][None, :]

    return pl.pallas_call(
        body,
        out_shape=jax.ShapeDtypeStruct((N, 32), jnp.float32),
    )(op2, dp2, b2)


def kernel(x, edge_index, W1, a1_src, a1_dst, b1, W2, a2_src, a2_dst, b2):
    src = edge_index[0].reshape(NW, EPW)
    dst = edge_index[1].reshape(NW, EPW)
    src = jnp.pad(src, ((0, 0), (0, EPWP - EPW)), constant_values=N)
    dst = jnp.pad(dst, ((0, 0), (0, EPWP - EPW)), constant_values=N)
    src = src.reshape(NW, NCHP, CP)
    dst = dst.reshape(NW, NCHP, CP)

    h1, sd1 = _tc_front(x, W1, a1_src, a1_dst)
    dp1, op1 = _sc_layer1(src, dst, sd1[:, 0], sd1[:, 1], h1)
    h2, sd2 = _tc_mid(op1, dp1, b1, W2, a2_src, a2_dst)
    dp2, op2 = _sc_layer2(src, dst, sd2[:, 0], sd2[:, 1], h2)
    return _tc_back(op2, dp2, b2)


# speedup vs baseline: 1.1210x; 1.1210x over previous
import functools

import jax
import jax.numpy as jnp
from jax import lax
from jax.experimental import pallas as pl
from jax.experimental.pallas import tpu as pltpu
from jax.experimental.pallas import tpu_sc as plsc

N = 10000
E = 320000
NC, NS = 2, 16
NW = NC * NS
EPW = E // NW
CP = 128
EPWP = 10240
NCHP = EPWP // CP
NP = 10240
RPT = NP // NS
NBUF = 4
NOUT = 2


def _sc_layer(D):
    D2 = D // 2
    mesh = plsc.VectorSubcoreMesh(core_axis_name="c", subcore_axis_name="s")

    @functools.partial(
        pl.kernel,
        out_type=[
            jax.ShapeDtypeStruct((NC, NP), jnp.float32),
            jax.ShapeDtypeStruct((NC, NP, D2), jnp.float32),
        ],
        mesh=mesh,
        compiler_params=pltpu.CompilerParams(
            needs_layout_passes=False, use_tc_tiling_on_sc=False),
        scratch_types=[
            pltpu.VMEM((NCHP, CP), jnp.int32),
            pltpu.VMEM((NCHP, CP), jnp.int32),
            pltpu.VMEM((NP,), jnp.float32),
            pltpu.VMEM((NP,), jnp.float32),
            pltpu.VMEM((EPWP,), jnp.float32),
            pltpu.VMEM((NBUF, CP, D2), jnp.float32),
            pltpu.VMEM((NOUT, CP, D2), jnp.float32),
            pltpu.VMEM((RPT,), jnp.float32),
            pltpu.VMEM_SHARED((NP,), jnp.float32),
            pltpu.VMEM_SHARED((NP, D2), jnp.float32),
            pltpu.VMEM_SHARED((NP, D2), jnp.float32),
            pltpu.SemaphoreType.DMA((NBUF,)),
            pltpu.SemaphoreType.DMA((NOUT,)),
            pltpu.SemaphoreType.DMA((NOUT,)),
            pltpu.SemaphoreType.DMA,
        ],
    )
    def k(src_hbm, dst_hbm, s_hbm, d_hbm, h_hbm, den_out, out_out,
          srcv, dstv, sv, dv, exv, rows, rows_out, zrow, dacc, oacc, hbuf,
          gsem, ssem, dsem, hsem):
        cid = lax.axis_index("c")
        sid = lax.axis_index("s")

        zero = jnp.zeros((16,), jnp.float32)
        zidx = jnp.zeros((16,), jnp.int32)
        oidx = jnp.ones((16,), jnp.int32)

        pltpu.async_copy(s_hbm, sv, gsem.at[2])
        pltpu.async_copy(d_hbm, dv, gsem.at[3])
        pltpu.async_copy(h_hbm.at[cid, pl.ds(sid * RPT, RPT)],
                         hbuf.at[pl.ds(sid * RPT, RPT)], hsem)

        @pl.loop(0, RPT // 16, unroll=8)
        def _(i):
            zrow[pl.ds(i * 16, 16)] = zero

        @pl.loop(0, CP, unroll=4)
        def _(r):
            for t in range(D2 // 16):
                rows_out[0, r, pl.ds(t * 16, 16)] = zero

        pltpu.sync_copy(zrow, dacc.at[pl.ds(sid * RPT, RPT)])
        for t in range(RPT // CP):
            pltpu.sync_copy(rows_out.at[0],
                            oacc.at[pl.ds(sid * RPT + t * CP, CP)])

        pltpu.make_async_copy(s_hbm, sv, gsem.at[2]).wait()
        pltpu.make_async_copy(d_hbm, dv, gsem.at[3]).wait()
        pltpu.make_async_copy(h_hbm.at[cid, pl.ds(sid * RPT, RPT)],
                              hbuf.at[pl.ds(sid * RPT, RPT)], hsem).wait()
        plsc.subcore_barrier()

        for p in range(2):
            wp = sid * 2 + p
            do_den = cid == p
            pltpu.async_copy(src_hbm.at[wp], srcv, gsem.at[0])
            pltpu.async_copy(dst_hbm.at[wp], dstv, gsem.at[1])
            pltpu.make_async_copy(src_hbm.at[wp], srcv, gsem.at[0]).wait()
            pltpu.make_async_copy(dst_hbm.at[wp], dstv, gsem.at[1]).wait()

            @plsc.parallel_loop(0, EPWP // 16, unroll=8)
            def _(v):
                row = v >> 3
                col = (v & 7) * 16
                si = srcv[row, pl.ds(col, 16)]
                di = dstv[row, pl.ds(col, 16)]
                e = plsc.load_gather(sv, [si]) + plsc.load_gather(dv, [di])
                e = jnp.where(e >= 0.0, e, 0.2 * e)
                exv[pl.ds(v * 16, 16)] = jnp.exp(e)

            for c in range(2):
                pltpu.async_copy(hbuf.at[srcv.at[c]], rows.at[c], gsem.at[c])

            @pl.loop(0, NCHP, step=NBUF)
            def _(cbase):
                for b in range(NBUF):
                    c = cbase + b
                    b2 = b % NOUT
                    bn = (b + 2) % NBUF

                    @pl.when(c < NCHP - 2)
                    def _():
                        pltpu.async_copy(
                            hbuf.at[srcv.at[c + 2]], rows.at[bn],
                            gsem.at[bn])

                    pltpu.make_async_copy(
                        hbuf.at[srcv.at[c]], rows.at[b], gsem.at[b]).wait()

                    @pl.when(c >= NOUT)
                    def _():
                        pltpu.make_async_copy(
                            rows_out.at[b2], oacc.at[dstv.at[c - NOUT]],
                            ssem.at[b2]).wait()

                        @pl.when(do_den)
                        def _():
                            pltpu.make_async_copy(
                                exv.at[pl.ds((c - NOUT) * CP, CP)],
                                dacc.at[dstv.at[c - NOUT]],
                                dsem.at[b2]).wait()

                    @plsc.parallel_loop(0, CP, unroll=8)
                    def _(r):
                        aidx = jnp.full((16,), c * CP + r, jnp.int32)
                        a = plsc.load_gather(exv, [aidx])
                        for t in range(D2 // 16):
                            rows_out[b2, r, pl.ds(t * 16, 16)] = (
                                rows[b, r, pl.ds(t * 16, 16)] * a)

                    pltpu.async_copy(
                        rows_out.at[b2], oacc.at[dstv.at[c]], ssem.at[b2],
                        add=True)

                    @pl.when(do_den)
                    def _():
                        pltpu.async_copy(
                            exv.at[pl.ds(c * CP, CP)], dacc.at[dstv.at[c]],
                            dsem.at[b2], add=True)

            for c in range(NCHP - NOUT, NCHP):
                b2 = c % NOUT
                pltpu.make_async_copy(
                    rows_out.at[b2], oacc.at[dstv.at[c]], ssem.at[b2]).wait()

                @pl.when(do_den)
                def _():
                    pltpu.make_async_copy(
                        exv.at[pl.ds(c * CP, CP)], dacc.at[dstv.at[c]],
                        dsem.at[b2]).wait()

        plsc.subcore_barrier()

        pltpu.sync_copy(dacc.at[pl.ds(sid * RPT, RPT)],
                        den_out.at[cid, pl.ds(sid * RPT, RPT)])
        pltpu.sync_copy(oacc.at[pl.ds(sid * RPT, RPT)],
                        out_out.at[cid, pl.ds(sid * RPT, RPT)])

    return k


_sc_layer1 = _sc_layer(64)
_sc_layer2 = _sc_layer(32)


def _tc_front(x, W1, a1s, a1d):
    def body(x_ref, w_ref, as_ref, ad_ref, h_ref, sd_ref):
        h = jnp.dot(x_ref[...], w_ref[...], preferred_element_type=jnp.float32)
        hp = jnp.concatenate(
            [h, jnp.zeros((NP - N, 64), jnp.float32)], axis=0)
        h_ref[0] = hp[:, :32]
        h_ref[1] = hp[:, 32:]
        a = jnp.concatenate(
            [as_ref[...][:, None], ad_ref[...][:, None]], axis=1)
        sd_ref[...] = jnp.dot(hp, a, preferred_element_type=jnp.float32)

    return pl.pallas_call(
        body,
        out_shape=[
            jax.ShapeDtypeStruct((NC, NP, 32), jnp.float32),
            jax.ShapeDtypeStruct((NP, 2), jnp.float32),
        ],
    )(x, W1, a1s, a1d)


def _tc_mid(op1, dp1, b1, W2, a2s, a2d):
    def body(op_ref, dp_ref, b_ref, w_ref, as_ref, ad_ref, h_ref, sd_ref):
        acc = jnp.concatenate([op_ref[0], op_ref[1]], axis=1)
        den = dp_ref[0] + dp_ref[1]
        rden = 1.0 / (den + 1e-16)
        xx = acc * rden[:, None] + b_ref[...][None, :]
        xx = jnp.where(xx > 0.0, xx, jnp.exp(xx) - 1.0)
        h2 = jnp.dot(xx, w_ref[...], preferred_element_type=jnp.float32)
        h_ref[0] = h2[:, :16]
        h_ref[1] = h2[:, 16:]
        a = jnp.concatenate(
            [as_ref[...][:, None], ad_ref[...][:, None]], axis=1)
        sd_ref[...] = jnp.dot(h2, a, preferred_element_type=jnp.float32)

    return pl.pallas_call(
        body,
        out_shape=[
            jax.ShapeDtypeStruct((NC, NP, 16), jnp.float32),
            jax.ShapeDtypeStruct((NP, 2), jnp.float32),
        ],
    )(op1, dp1, b1, W2, a2s, a2d)


def _tc_back(op2, dp2, b2):
    def body(op_ref, dp_ref, b_ref, o_ref):
        acc = jnp.concatenate([op_ref[0], op_ref[1]], axis=1)[:N]
        den = (dp_ref[0] + dp_ref[1])[:N]
        rden = 1.0 / (den + 1e-16)
        o_ref[...] = acc * rden[:, None] + b_ref[...][None, :]

    return pl.pallas_call(
        body,
        out_shape=jax.ShapeDtypeStruct((N, 32), jnp.float32),
    )(op2, dp2, b2)


def kernel(x, edge_index, W1, a1_src, a1_dst, b1, W2, a2_src, a2_dst, b2):
    src = edge_index[0].reshape(NW, EPW)
    dst = edge_index[1].reshape(NW, EPW)
    src = jnp.pad(src, ((0, 0), (0, EPWP - EPW)), constant_values=N)
    dst = jnp.pad(dst, ((0, 0), (0, EPWP - EPW)), constant_values=N)
    src = src.reshape(NW, NCHP, CP)
    dst = dst.reshape(NW, NCHP, CP)

    h1, sd1 = _tc_front(x, W1, a1_src, a1_dst)
    dp1, op1 = _sc_layer1(src, dst, sd1[:, 0], sd1[:, 1], h1)
    h2, sd2 = _tc_mid(op1, dp1, b1, W2, a2_src, a2_dst)
    dp2, op2 = _sc_layer2(src, dst, sd2[:, 0], sd2[:, 1], h2)
    return _tc_back(op2, dp2, b2)


# speedup vs baseline: 1.2158x; 1.0846x over previous
import functools

import jax
import jax.numpy as jnp
from jax import lax
from jax.experimental import pallas as pl
from jax.experimental.pallas import tpu as pltpu
from jax.experimental.pallas import tpu_sc as plsc

N = 10000
E = 320000
NC, NS = 2, 16
NW = NC * NS
EPW = E // NW
CP = 128
EPWP = 10240
NCHP = EPWP // CP
NP = 10240
RPT = NP // NS
NBUF = 4
NOUT = 2


def _sc_layer(D, hdtype=jnp.float32):
    D2 = D // 2
    bf = hdtype == jnp.bfloat16
    mesh = plsc.VectorSubcoreMesh(core_axis_name="c", subcore_axis_name="s")

    @functools.partial(
        pl.kernel,
        out_type=[
            jax.ShapeDtypeStruct((NC, NP), jnp.float32),
            jax.ShapeDtypeStruct((NC, NP, D2), jnp.float32),
        ],
        mesh=mesh,
        compiler_params=pltpu.CompilerParams(
            needs_layout_passes=False, use_tc_tiling_on_sc=False),
        scratch_types=[
            pltpu.VMEM((NCHP, CP), jnp.int32),
            pltpu.VMEM((NCHP, CP), jnp.int32),
            pltpu.VMEM((NP,), jnp.float32),
            pltpu.VMEM((NP,), jnp.float32),
            pltpu.VMEM((EPWP,), jnp.float32),
            pltpu.VMEM((NBUF, CP, D2), hdtype),
            pltpu.VMEM((NOUT, CP, D2), jnp.float32),
            pltpu.VMEM((RPT,), jnp.float32),
            pltpu.VMEM_SHARED((NP,), jnp.float32),
            pltpu.VMEM_SHARED((NP, D2), jnp.float32),
            pltpu.VMEM_SHARED((NP, D2), hdtype),
            pltpu.SemaphoreType.DMA((NBUF,)),
            pltpu.SemaphoreType.DMA((NOUT,)),
            pltpu.SemaphoreType.DMA((NOUT,)),
            pltpu.SemaphoreType.DMA,
        ],
    )
    def k(src_hbm, dst_hbm, s_hbm, d_hbm, h_hbm, den_out, out_out,
          srcv, dstv, sv, dv, exv, rows, rows_out, zrow, dacc, oacc, hbuf,
          gsem, ssem, dsem, hsem):
        cid = lax.axis_index("c")
        sid = lax.axis_index("s")

        zero = jnp.zeros((16,), jnp.float32)
        zidx = jnp.zeros((16,), jnp.int32)
        oidx = jnp.ones((16,), jnp.int32)

        pltpu.async_copy(s_hbm, sv, gsem.at[2])
        pltpu.async_copy(d_hbm, dv, gsem.at[3])
        pltpu.async_copy(h_hbm.at[cid, pl.ds(sid * RPT, RPT)],
                         hbuf.at[pl.ds(sid * RPT, RPT)], hsem)

        @pl.loop(0, RPT // 16, unroll=8)
        def _(i):
            zrow[pl.ds(i * 16, 16)] = zero

        @pl.loop(0, CP, unroll=4)
        def _(r):
            for t in range(D2 // 16):
                rows_out[0, r, pl.ds(t * 16, 16)] = zero

        pltpu.sync_copy(zrow, dacc.at[pl.ds(sid * RPT, RPT)])
        for t in range(RPT // CP):
            pltpu.sync_copy(rows_out.at[0],
                            oacc.at[pl.ds(sid * RPT + t * CP, CP)])

        pltpu.make_async_copy(s_hbm, sv, gsem.at[2]).wait()
        pltpu.make_async_copy(d_hbm, dv, gsem.at[3]).wait()
        pltpu.make_async_copy(h_hbm.at[cid, pl.ds(sid * RPT, RPT)],
                              hbuf.at[pl.ds(sid * RPT, RPT)], hsem).wait()
        plsc.subcore_barrier()

        for p in range(2):
            wp = sid * 2 + p
            do_den = cid == p
            pltpu.async_copy(src_hbm.at[wp], srcv, gsem.at[0])
            pltpu.async_copy(dst_hbm.at[wp], dstv, gsem.at[1])
            pltpu.make_async_copy(src_hbm.at[wp], srcv, gsem.at[0]).wait()
            pltpu.make_async_copy(dst_hbm.at[wp], dstv, gsem.at[1]).wait()

            @plsc.parallel_loop(0, EPWP // 16, unroll=8)
            def _(v):
                row = v >> 3
                col = (v & 7) * 16
                si = srcv[row, pl.ds(col, 16)]
                di = dstv[row, pl.ds(col, 16)]
                e = plsc.load_gather(sv, [si]) + plsc.load_gather(dv, [di])
                e = jnp.where(e >= 0.0, e, 0.2 * e)
                exv[pl.ds(v * 16, 16)] = jnp.exp(e)

            for c in range(2):
                pltpu.async_copy(hbuf.at[srcv.at[c]], rows.at[c], gsem.at[c])

            @pl.loop(0, NCHP, step=NBUF)
            def _(cbase):
                for b in range(NBUF):
                    c = cbase + b
                    b2 = b % NOUT
                    bn = (b + 2) % NBUF

                    @pl.when(c < NCHP - 2)
                    def _():
                        pltpu.async_copy(
                            hbuf.at[srcv.at[c + 2]], rows.at[bn],
                            gsem.at[bn])

                    pltpu.make_async_copy(
                        hbuf.at[srcv.at[c]], rows.at[b], gsem.at[b]).wait()

                    @pl.when(c >= NOUT)
                    def _():
                        pltpu.make_async_copy(
                            rows_out.at[b2], oacc.at[dstv.at[c - NOUT]],
                            ssem.at[b2]).wait()

                        @pl.when(do_den)
                        def _():
                            pltpu.make_async_copy(
                                exv.at[pl.ds((c - NOUT) * CP, CP)],
                                dacc.at[dstv.at[c - NOUT]],
                                dsem.at[b2]).wait()

                    if bf:
                        ecol = jnp.arange(16, dtype=jnp.int32) * 2

                        @plsc.parallel_loop(0, CP, unroll=8)
                        def _(r):
                            aidx = jnp.full((16,), c * CP + r, jnp.int32)
                            a = plsc.load_gather(exv, [aidx])
                            rsp = jnp.full((16,), r, jnp.int32)
                            for t in range(D2 // 32):
                                w = rows[b, r, pl.ds(t * 32, 32)]
                                ev, od = plsc.unpack(
                                    w, format=plsc.PackFormat.INTERLEAVED)
                                col = ecol + t * 32
                                plsc.store_scatter(
                                    rows_out.at[b2], [rsp, col], ev * a)
                                plsc.store_scatter(
                                    rows_out.at[b2], [rsp, col + 1], od * a)
                    else:
                        @plsc.parallel_loop(0, CP, unroll=8)
                        def _(r):
                            aidx = jnp.full((16,), c * CP + r, jnp.int32)
                            a = plsc.load_gather(exv, [aidx])
                            for t in range(D2 // 16):
                                rows_out[b2, r, pl.ds(t * 16, 16)] = (
                                    rows[b, r, pl.ds(t * 16, 16)] * a)

                    pltpu.async_copy(
                        rows_out.at[b2], oacc.at[dstv.at[c]], ssem.at[b2],
                        add=True)

                    @pl.when(do_den)
                    def _():
                        pltpu.async_copy(
                            exv.at[pl.ds(c * CP, CP)], dacc.at[dstv.at[c]],
                            dsem.at[b2], add=True)

            for c in range(NCHP - NOUT, NCHP):
                b2 = c % NOUT
                pltpu.make_async_copy(
                    rows_out.at[b2], oacc.at[dstv.at[c]], ssem.at[b2]).wait()

                @pl.when(do_den)
                def _():
                    pltpu.make_async_copy(
                        exv.at[pl.ds(c * CP, CP)], dacc.at[dstv.at[c]],
                        dsem.at[b2]).wait()

        plsc.subcore_barrier()

        pltpu.sync_copy(dacc.at[pl.ds(sid * RPT, RPT)],
                        den_out.at[cid, pl.ds(sid * RPT, RPT)])
        pltpu.sync_copy(oacc.at[pl.ds(sid * RPT, RPT)],
                        out_out.at[cid, pl.ds(sid * RPT, RPT)])

    return k


_sc_layer1 = _sc_layer(64, jnp.bfloat16)
_sc_layer2 = _sc_layer(32)


def _tc_front(x, W1, a1s, a1d):
    def body(x_ref, w_ref, as_ref, ad_ref, h_ref, sd_ref):
        h = jnp.dot(x_ref[...], w_ref[...], preferred_element_type=jnp.float32)
        hp = jnp.concatenate(
            [h, jnp.zeros((NP - N, 64), jnp.float32)], axis=0)
        h_ref[0] = hp[:, :32].astype(jnp.bfloat16)
        h_ref[1] = hp[:, 32:].astype(jnp.bfloat16)
        a = jnp.concatenate(
            [as_ref[...][:, None], ad_ref[...][:, None]], axis=1)
        sd_ref[...] = jnp.dot(hp, a, preferred_element_type=jnp.float32)

    return pl.pallas_call(
        body,
        out_shape=[
            jax.ShapeDtypeStruct((NC, NP, 32), jnp.bfloat16),
            jax.ShapeDtypeStruct((NP, 2), jnp.float32),
        ],
    )(x, W1, a1s, a1d)


def _tc_mid(op1, dp1, b1, W2, a2s, a2d):
    def body(op_ref, dp_ref, b_ref, w_ref, as_ref, ad_ref, h_ref, sd_ref):
        acc = jnp.concatenate([op_ref[0], op_ref[1]], axis=1)
        den = dp_ref[0] + dp_ref[1]
        rden = 1.0 / (den + 1e-16)
        xx = acc * rden[:, None] + b_ref[...][None, :]
        xx = jnp.where(xx > 0.0, xx, jnp.exp(xx) - 1.0)
        h2 = jnp.dot(xx, w_ref[...], preferred_element_type=jnp.float32)
        h_ref[0] = h2[:, :16]
        h_ref[1] = h2[:, 16:]
        a = jnp.concatenate(
            [as_ref[...][:, None], ad_ref[...][:, None]], axis=1)
        sd_ref[...] = jnp.dot(h2, a, preferred_element_type=jnp.float32)

    return pl.pallas_call(
        body,
        out_shape=[
            jax.ShapeDtypeStruct((NC, NP, 16), jnp.float32),
            jax.ShapeDtypeStruct((NP, 2), jnp.float32),
        ],
    )(op1, dp1, b1, W2, a2s, a2d)


def _tc_back(op2, dp2, b2):
    def body(op_ref, dp_ref, b_ref, o_ref):
        acc = jnp.concatenate([op_ref[0], op_ref[1]], axis=1)[:N]
        den = (dp_ref[0] + dp_ref[1])[:N]
        rden = 1.0 / (den + 1e-16)
        o_ref[...] = acc * rden[:, None] + b_ref[...][None, :]

    return pl.pallas_call(
        body,
        out_shape=jax.ShapeDtypeStruct((N, 32), jnp.float32),
    )(op2, dp2, b2)


def kernel(x, edge_index, W1, a1_src, a1_dst, b1, W2, a2_src, a2_dst, b2):
    src = edge_index[0].reshape(NW, EPW)
    dst = edge_index[1].reshape(NW, EPW)
    src = jnp.pad(src, ((0, 0), (0, EPWP - EPW)), constant_values=N)
    dst = jnp.pad(dst, ((0, 0), (0, EPWP - EPW)), constant_values=N)
    src = src.reshape(NW, NCHP, CP)
    dst = dst.reshape(NW, NCHP, CP)

    h1, sd1 = _tc_front(x, W1, a1_src, a1_dst)
    dp1, op1 = _sc_layer1(src, dst, sd1[:, 0], sd1[:, 1], h1)
    h2, sd2 = _tc_mid(op1, dp1, b1, W2, a2_src, a2_dst)
    dp2, op2 = _sc_layer2(src, dst, sd2[:, 0], sd2[:, 1], h2)
    return _tc_back(op2, dp2, b2)
